# baseline (speedup 1.0000x reference)
import functools

import jax
import jax.numpy as jnp
from jax import lax
from jax.experimental import pallas as pl
from jax.experimental.pallas import tpu as pltpu
from jax.experimental.pallas import tpu_sc as plsc

NC = 2
NS = 16
NW = NC * NS
CHUNK = 128
BLK = 128

_P = jax.lax.Precision.HIGHEST


def _sc_mesh():
    return plsc.VectorSubcoreMesh(
        core_axis_name="c", subcore_axis_name="s", num_cores=NC,
        num_subcores=NS)


def _deg_body(npt, ch, sp_ref, rp_ref, ones_hbm, zero_hbm, out_ref,
              idx_s_v, idx_r_v, ones_v, degs_sp, degr_sp):
    c = lax.axis_index("c")
    s = lax.axis_index("s")
    wid = c * NS + s
    rows = pl.ds(s * npt, npt)
    pltpu.sync_copy(zero_hbm.at[rows], degs_sp.at[rows])
    pltpu.sync_copy(zero_hbm.at[rows], degr_sp.at[rows])
    pltpu.sync_copy(ones_hbm, ones_v)
    pltpu.sync_copy(sp_ref.at[wid], idx_s_v)
    pltpu.sync_copy(rp_ref.at[wid], idx_r_v)
    plsc.subcore_barrier()

    def body(j, carry):
        pltpu.sync_copy(ones_v, degs_sp.at[idx_s_v.at[j]], add=True)
        pltpu.sync_copy(ones_v, degr_sp.at[idx_r_v.at[j]], add=True)
        return carry

    lax.fori_loop(0, ch, body, 0)
    plsc.subcore_barrier()
    pltpu.sync_copy(degs_sp.at[rows], out_ref.at[c, 0, rows])
    pltpu.sync_copy(degr_sp.at[rows], out_ref.at[c, 1, rows])


def _prop_body(npt, ch, nb, table_hbm, sp_ref, rp_ref, zero_hbm, out_ref,
               idx_s_v, idx_r_v, rows_v, agg_sp, gsem0, gsem1, ssem0, ssem1):
    c = lax.axis_index("c")
    s = lax.axis_index("s")
    wid = c * NS + s
    rows = pl.ds(s * npt, npt)
    gsem = (gsem0, gsem1)
    ssem = (ssem0, ssem1)
    pltpu.sync_copy(zero_hbm.at[rows], agg_sp.at[rows])
    pltpu.sync_copy(sp_ref.at[wid], idx_s_v)
    pltpu.sync_copy(rp_ref.at[wid], idx_r_v)
    plsc.subcore_barrier()

    def issue_gathers(p, base):
        for b in range(nb):
            pltpu.async_copy(table_hbm.at[idx_s_v.at[base + b]],
                             rows_v.at[p, b], gsem[p])

    def drain_gathers(p):
        for b in range(nb):
            pltpu.make_async_copy(table_hbm.at[idx_s_v.at[0]],
                                  rows_v.at[p, b], gsem[p]).wait()

    def issue_scatters(p, base):
        for b in range(nb):
            pltpu.async_copy(rows_v.at[p, b], agg_sp.at[idx_r_v.at[base + b]],
                             ssem[p], add=True)

    def drain_scatters(p):
        for b in range(nb):
            pltpu.make_async_copy(rows_v.at[p, b], agg_sp.at[idx_r_v.at[0]],
                                  ssem[p]).wait()

    nk = ch // (2 * nb)
    issue_gathers(0, 0)

    def body(k, carry):
        base0 = k * 2 * nb
        base1 = base0 + nb
        drain_gathers(0)

        @pl.when(k > 0)
        def _():
            drain_scatters(1)

        issue_gathers(1, base1)
        issue_scatters(0, base0)
        drain_gathers(1)
        drain_scatters(0)

        @pl.when(k + 1 < nk)
        def _():
            issue_gathers(0, base0 + 2 * nb)

        issue_scatters(1, base1)
        return carry

    lax.fori_loop(0, nk, body, 0)
    drain_scatters(1)
    plsc.subcore_barrier()
    pltpu.sync_copy(agg_sp.at[rows], out_ref.at[c, rows])


def _scales(deg_blk):
    ds_ = deg_blk[:, 0:1] + deg_blk[:, 2:3]
    dr_ = deg_blk[:, 1:2] + deg_blk[:, 3:4]
    ss = lax.rsqrt(jnp.maximum(ds_, 1.0))
    sr = lax.rsqrt(jnp.maximum(dr_, 1.0))
    return ss, sr


def _softmax(a):
    e = jnp.exp(a - jnp.max(a, axis=-1, keepdims=True))
    return e / jnp.sum(e, axis=-1, keepdims=True)


def _pad_cols(t, w):
    return jnp.concatenate(
        [t, jnp.zeros((t.shape[0], w - t.shape[1]), jnp.float32)], axis=1)


def _prep1_body(hp, x_ref, deg_ref, w_ref, b_ref, o_ref):
    ss, _ = _scales(deg_ref[...])
    a = jnp.maximum(
        jnp.dot(x_ref[...], w_ref[...], preferred_element_type=jnp.float32,
                precision=_P) + b_ref[...], 0.0)
    o_ref[...] = _pad_cols(_softmax(a) * ss, hp)


def _prep2_body(h, hp, agg_ref, deg_ref, w_ref, b_ref, o_ref):
    ss, sr = _scales(deg_ref[...])
    h1 = (agg_ref[0] + agg_ref[1])[:, :h] * sr
    a = jnp.maximum(
        jnp.dot(h1, w_ref[...], preferred_element_type=jnp.float32,
                precision=_P) + b_ref[...], 0.0)
    o_ref[...] = _pad_cols(_softmax(a) * ss, hp)


def _final_body(h, agg_ref, deg_ref, x_ref, nz_ref, wmu_ref, bmu_ref,
                wlv_ref, blv_ref, wd1_ref, bd1_ref, wd2_ref, bd2_ref,
                xs_ref, mu_ref, lv_ref):
    _, sr = _scales(deg_ref[...])
    h2 = (agg_ref[0] + agg_ref[1])[:, :h] * sr
    xb = x_ref[...]
    wmu = wmu_ref[...]
    wlv = wlv_ref[...]
    dot = functools.partial(jnp.dot, preferred_element_type=jnp.float32,
                            precision=_P)
    mu = dot(h2, wmu[:h]) + dot(xb, wmu[h:]) + bmu_ref[...]
    lv = dot(h2, wlv[:h]) + dot(xb, wlv[h:]) + blv_ref[...]
    sigma = 0.0001 + jnp.exp(0.5 * lv)
    z = mu + sigma * nz_ref[...]
    d = jnp.maximum(dot(z, wd1_ref[...]) + bd1_ref[...], 0.0)
    xs_ref[...] = dot(d, wd2_ref[...]) + bd2_ref[...]
    mu_ref[...] = mu
    lv_ref[...] = lv


def kernel(x, edge_index, noise, W1, b1, W2, b2, Wmu, bmu, Wlv, blv,
           Wd1, bd1, Wd2, bd2):
    f32 = jnp.float32
    n, d_feat = x.shape
    e = edge_index.shape[1]
    h = W1.shape[1]
    z_dim = Wmu.shape[1]
    dec_h = Wd1.shape[1]
    expr = Wd2.shape[1]

    hp = 32
    dw = 16
    nb = 4
    ch = 2 * nb * -(-e // (NW * CHUNK * 2 * nb))
    e_pad = NW * ch * CHUNK
    npad = BLK * (-(-(n + 1) // BLK))
    npt = npad // NS

    senders = jnp.concatenate(
        [edge_index[0], jnp.full((e_pad - e,), n, jnp.int32)]
    ).reshape(NW, ch, CHUNK)
    receivers = jnp.concatenate(
        [edge_index[1], jnp.full((e_pad - e,), n, jnp.int32)]
    ).reshape(NW, ch, CHUNK)

    zero_nd = jnp.zeros((npad, dw), f32)
    zero_nh = jnp.zeros((npad, hp), f32)
    ones_cd = jnp.ones((CHUNK, dw), f32)

    deg_part = pl.kernel(
        functools.partial(_deg_body, npt, ch),
        out_type=jax.ShapeDtypeStruct((NC, 2, npad, dw), f32),
        mesh=_sc_mesh(),
        scratch_types=[
            pltpu.VMEM((ch, CHUNK), jnp.int32),
            pltpu.VMEM((ch, CHUNK), jnp.int32),
            pltpu.VMEM((CHUNK, dw), f32),
            pltpu.VMEM_SHARED((npad, dw), f32),
            pltpu.VMEM_SHARED((npad, dw), f32),
        ],
        compiler_params=pltpu.CompilerParams(use_tc_tiling_on_sc=False),
    )(senders, receivers, ones_cd, zero_nd)
    deg_cols = deg_part[:, :, :, 0].reshape(NC * 2, npad).T

    grid = (npad // BLK,)
    row_spec = lambda w: pl.BlockSpec((BLK, w), lambda i: (i, 0))
    full_spec = lambda a, b: pl.BlockSpec((a, b), lambda i: (0, 0))
    agg_spec = pl.BlockSpec((NC, BLK, hp), lambda i: (0, i, 0))

    table1 = pl.pallas_call(
        functools.partial(_prep1_body, hp),
        grid=grid,
        in_specs=[row_spec(d_feat), row_spec(4), full_spec(d_feat, h),
                  full_spec(1, h)],
        out_specs=row_spec(hp),
        out_shape=jax.ShapeDtypeStruct((npad, hp), f32),
    )(x, deg_cols, W1, b1.reshape(1, h))

    def propagate(table):
        return pl.kernel(
            functools.partial(_prop_body, npt, ch, nb),
            out_type=jax.ShapeDtypeStruct((NC, npad, hp), f32),
            mesh=_sc_mesh(),
            scratch_types=[
                pltpu.VMEM((ch, CHUNK), jnp.int32),
                pltpu.VMEM((ch, CHUNK), jnp.int32),
                pltpu.VMEM((2, nb, CHUNK, hp), f32),
                pltpu.VMEM_SHARED((npad, hp), f32),
                pltpu.SemaphoreType.DMA,
                pltpu.SemaphoreType.DMA,
                pltpu.SemaphoreType.DMA,
                pltpu.SemaphoreType.DMA,
            ],
            compiler_params=pltpu.CompilerParams(use_tc_tiling_on_sc=False),
        )(table, senders, receivers, zero_nh)

    agg1 = propagate(table1)

    table2 = pl.pallas_call(
        functools.partial(_prep2_body, h, hp),
        grid=grid,
        in_specs=[agg_spec, row_spec(4), full_spec(h, h), full_spec(1, h)],
        out_specs=row_spec(hp),
        out_shape=jax.ShapeDtypeStruct((npad, hp), f32),
    )(agg1, deg_cols, W2, b2.reshape(1, h))

    agg2 = propagate(table2)

    xs, mu, lv = pl.pallas_call(
        functools.partial(_final_body, h),
        grid=grid,
        in_specs=[agg_spec, row_spec(4), row_spec(d_feat), row_spec(z_dim),
                  full_spec(h + d_feat, z_dim), full_spec(1, z_dim),
                  full_spec(h + d_feat, z_dim), full_spec(1, z_dim),
                  full_spec(z_dim, dec_h), full_spec(1, dec_h),
                  full_spec(dec_h, expr), full_spec(1, expr)],
        out_specs=[row_spec(expr), row_spec(z_dim), row_spec(z_dim)],
        out_shape=[
            jax.ShapeDtypeStruct((npad, expr), f32),
            jax.ShapeDtypeStruct((npad, z_dim), f32),
            jax.ShapeDtypeStruct((npad, z_dim), f32),
        ],
    )(agg2, deg_cols, x, noise, Wmu, bmu.reshape(1, z_dim),
      Wlv, blv.reshape(1, z_dim), Wd1, bd1.reshape(1, dec_h),
      Wd2, bd2.reshape(1, expr))

    return (xs[:n], mu[:n], lv[:n])


# speedup vs baseline: 1.4858x; 1.4858x over previous
import functools

import jax
import jax.numpy as jnp
from jax import lax
from jax.experimental import pallas as pl
from jax.experimental.pallas import tpu as pltpu
from jax.experimental.pallas import tpu_sc as plsc

NC = 2
NS = 16
NW = NC * NS
CHUNK = 128
BLK = 128

_P = jax.lax.Precision.HIGHEST


def _sc_mesh():
    return plsc.VectorSubcoreMesh(
        core_axis_name="c", subcore_axis_name="s", num_cores=NC,
        num_subcores=NS)


def _deg_body(npt, ch, sp_ref, rp_ref, ones_hbm, zero_hbm, out_ref,
              idx_s_v, idx_r_v, ones_v, degs_sp, degr_sp):
    c = lax.axis_index("c")
    s = lax.axis_index("s")
    wid = c * NS + s
    rows = pl.ds(s * npt, npt)
    pltpu.sync_copy(zero_hbm.at[rows], degs_sp.at[rows])
    pltpu.sync_copy(zero_hbm.at[rows], degr_sp.at[rows])
    pltpu.sync_copy(ones_hbm, ones_v)
    pltpu.sync_copy(sp_ref.at[wid], idx_s_v)
    pltpu.sync_copy(rp_ref.at[wid], idx_r_v)
    plsc.subcore_barrier()

    def body(j, carry):
        pltpu.sync_copy(ones_v, degs_sp.at[idx_s_v.at[j]], add=True)
        pltpu.sync_copy(ones_v, degr_sp.at[idx_r_v.at[j]], add=True)
        return carry

    lax.fori_loop(0, ch, body, 0)
    plsc.subcore_barrier()
    pltpu.sync_copy(degs_sp.at[rows], out_ref.at[c, 0, rows])
    pltpu.sync_copy(degr_sp.at[rows], out_ref.at[c, 1, rows])


def _prop_body(npt, ch, nb, table_hbm, sp_ref, rp_ref, zero_hbm, out_ref,
               idx_s_v, idx_r_v, rows_v, agg_sp, gsem0, gsem1, ssem0, ssem1):
    c = lax.axis_index("c")
    s = lax.axis_index("s")
    wid = c * NS + s
    rows = pl.ds(s * npt, npt)
    gsem = (gsem0, gsem1)
    ssem = (ssem0, ssem1)
    pltpu.sync_copy(zero_hbm.at[rows], agg_sp.at[rows])
    pltpu.sync_copy(sp_ref.at[wid], idx_s_v)
    pltpu.sync_copy(rp_ref.at[wid], idx_r_v)
    plsc.subcore_barrier()

    def issue_gathers(p, base):
        for b in range(nb):
            pltpu.async_copy(table_hbm.at[idx_s_v.at[base + b]],
                             rows_v.at[p, b], gsem[p])

    def drain_gathers(p):
        for b in range(nb):
            pltpu.make_async_copy(table_hbm.at[idx_s_v.at[0]],
                                  rows_v.at[p, b], gsem[p]).wait()

    def issue_scatters(p, base):
        for b in range(nb):
            pltpu.async_copy(rows_v.at[p, b], agg_sp.at[idx_r_v.at[base + b]],
                             ssem[p], add=True)

    def drain_scatters(p):
        for b in range(nb):
            pltpu.make_async_copy(rows_v.at[p, b], agg_sp.at[idx_r_v.at[0]],
                                  ssem[p]).wait()

    nk = ch // (2 * nb)
    issue_gathers(0, 0)

    def body(k, carry):
        base0 = k * 2 * nb
        base1 = base0 + nb
        drain_gathers(0)

        @pl.when(k > 0)
        def _():
            drain_scatters(1)

        issue_gathers(1, base1)
        issue_scatters(0, base0)
        drain_gathers(1)
        drain_scatters(0)

        @pl.when(k + 1 < nk)
        def _():
            issue_gathers(0, base0 + 2 * nb)

        issue_scatters(1, base1)
        return carry

    lax.fori_loop(0, nk, body, 0)
    drain_scatters(1)
    plsc.subcore_barrier()
    pltpu.sync_copy(agg_sp.at[rows], out_ref.at[c, rows])


def _scales(deg_blk):
    ds_ = deg_blk[:, 0:1] + deg_blk[:, 2:3]
    dr_ = deg_blk[:, 1:2] + deg_blk[:, 3:4]
    ss = lax.rsqrt(jnp.maximum(ds_, 1.0))
    sr = lax.rsqrt(jnp.maximum(dr_, 1.0))
    return ss, sr


def _softmax(a):
    e = jnp.exp(a - jnp.max(a, axis=-1, keepdims=True))
    return e / jnp.sum(e, axis=-1, keepdims=True)


def _pad_cols(t, w):
    return jnp.concatenate(
        [t, jnp.zeros((t.shape[0], w - t.shape[1]), jnp.float32)], axis=1)


def _prep1_body(hp, x_ref, deg_ref, w_ref, b_ref, o_ref):
    ss, _ = _scales(deg_ref[...])
    a = jnp.maximum(
        jnp.dot(x_ref[...], w_ref[...], preferred_element_type=jnp.float32,
                precision=_P) + b_ref[...], 0.0)
    o_ref[...] = _pad_cols(_softmax(a) * ss, hp)


def _prep2_body(h, hp, agg_ref, deg_ref, w_ref, b_ref, o_ref):
    ss, sr = _scales(deg_ref[...])
    h1 = (agg_ref[0] + agg_ref[1])[:, :h] * sr
    a = jnp.maximum(
        jnp.dot(h1, w_ref[...], preferred_element_type=jnp.float32,
                precision=_P) + b_ref[...], 0.0)
    o_ref[...] = _pad_cols(_softmax(a) * ss, hp)


def _final_body(h, agg_ref, deg_ref, x_ref, nz_ref, wmu_ref, bmu_ref,
                wlv_ref, blv_ref, wd1_ref, bd1_ref, wd2_ref, bd2_ref,
                xs_ref, mu_ref, lv_ref):
    _, sr = _scales(deg_ref[...])
    h2 = (agg_ref[0] + agg_ref[1])[:, :h] * sr
    xb = x_ref[...]
    wmu = wmu_ref[...]
    wlv = wlv_ref[...]
    dot = functools.partial(jnp.dot, preferred_element_type=jnp.float32,
                            precision=_P)
    mu = dot(h2, wmu[:h]) + dot(xb, wmu[h:]) + bmu_ref[...]
    lv = dot(h2, wlv[:h]) + dot(xb, wlv[h:]) + blv_ref[...]
    sigma = 0.0001 + jnp.exp(0.5 * lv)
    z = mu + sigma * nz_ref[...]
    d = jnp.maximum(dot(z, wd1_ref[...]) + bd1_ref[...], 0.0)
    xs_ref[...] = dot(d, wd2_ref[...]) + bd2_ref[...]
    mu_ref[...] = mu
    lv_ref[...] = lv


def kernel(x, edge_index, noise, W1, b1, W2, b2, Wmu, bmu, Wlv, blv,
           Wd1, bd1, Wd2, bd2):
    f32 = jnp.float32
    n, d_feat = x.shape
    e = edge_index.shape[1]
    h = W1.shape[1]
    z_dim = Wmu.shape[1]
    dec_h = Wd1.shape[1]
    expr = Wd2.shape[1]

    hp = 32
    dw = 16
    nb = 4
    ch = 2 * nb * -(-e // (NW * CHUNK * 2 * nb))
    e_pad = NW * ch * CHUNK
    npad = BLK * (-(-(n + 1) // BLK))
    npt = npad // NS

    pad_ids = n + jnp.arange(e_pad - e, dtype=jnp.int32) % (npad - n)
    senders = jnp.concatenate(
        [edge_index[0], pad_ids]).reshape(NW, ch, CHUNK)
    receivers = jnp.concatenate(
        [edge_index[1], pad_ids]).reshape(NW, ch, CHUNK)

    zero_nd = jnp.zeros((npad, dw), f32)
    zero_nh = jnp.zeros((npad, hp), f32)
    ones_cd = jnp.ones((CHUNK, dw), f32)

    deg_part = pl.kernel(
        functools.partial(_deg_body, npt, ch),
        out_type=jax.ShapeDtypeStruct((NC, 2, npad, dw), f32),
        mesh=_sc_mesh(),
        scratch_types=[
            pltpu.VMEM((ch, CHUNK), jnp.int32),
            pltpu.VMEM((ch, CHUNK), jnp.int32),
            pltpu.VMEM((CHUNK, dw), f32),
            pltpu.VMEM_SHARED((npad, dw), f32),
            pltpu.VMEM_SHARED((npad, dw), f32),
        ],
        compiler_params=pltpu.CompilerParams(use_tc_tiling_on_sc=False),
    )(senders, receivers, ones_cd, zero_nd)
    deg_cols = deg_part[:, :, :, 0].reshape(NC * 2, npad).T

    grid = (npad // BLK,)
    row_spec = lambda w: pl.BlockSpec((BLK, w), lambda i: (i, 0))
    full_spec = lambda a, b: pl.BlockSpec((a, b), lambda i: (0, 0))
    agg_spec = pl.BlockSpec((NC, BLK, hp), lambda i: (0, i, 0))

    table1 = pl.pallas_call(
        functools.partial(_prep1_body, hp),
        grid=grid,
        in_specs=[row_spec(d_feat), row_spec(4), full_spec(d_feat, h),
                  full_spec(1, h)],
        out_specs=row_spec(hp),
        out_shape=jax.ShapeDtypeStruct((npad, hp), f32),
    )(x, deg_cols, W1, b1.reshape(1, h))

    def propagate(table):
        return pl.kernel(
            functools.partial(_prop_body, npt, ch, nb),
            out_type=jax.ShapeDtypeStruct((NC, npad, hp), f32),
            mesh=_sc_mesh(),
            scratch_types=[
                pltpu.VMEM((ch, CHUNK), jnp.int32),
                pltpu.VMEM((ch, CHUNK), jnp.int32),
                pltpu.VMEM((2, nb, CHUNK, hp), f32),
                pltpu.VMEM_SHARED((npad, hp), f32),
                pltpu.SemaphoreType.DMA,
                pltpu.SemaphoreType.DMA,
                pltpu.SemaphoreType.DMA,
                pltpu.SemaphoreType.DMA,
            ],
            compiler_params=pltpu.CompilerParams(use_tc_tiling_on_sc=False),
        )(table, senders, receivers, zero_nh)

    agg1 = propagate(table1)

    table2 = pl.pallas_call(
        functools.partial(_prep2_body, h, hp),
        grid=grid,
        in_specs=[agg_spec, row_spec(4), full_spec(h, h), full_spec(1, h)],
        out_specs=row_spec(hp),
        out_shape=jax.ShapeDtypeStruct((npad, hp), f32),
    )(agg1, deg_cols, W2, b2.reshape(1, h))

    agg2 = propagate(table2)

    xs, mu, lv = pl.pallas_call(
        functools.partial(_final_body, h),
        grid=grid,
        in_specs=[agg_spec, row_spec(4), row_spec(d_feat), row_spec(z_dim),
                  full_spec(h + d_feat, z_dim), full_spec(1, z_dim),
                  full_spec(h + d_feat, z_dim), full_spec(1, z_dim),
                  full_spec(z_dim, dec_h), full_spec(1, dec_h),
                  full_spec(dec_h, expr), full_spec(1, expr)],
        out_specs=[row_spec(expr), row_spec(z_dim), row_spec(z_dim)],
        out_shape=[
            jax.ShapeDtypeStruct((n, expr), f32),
            jax.ShapeDtypeStruct((n, z_dim), f32),
            jax.ShapeDtypeStruct((n, z_dim), f32),
        ],
    )(agg2, deg_cols, x, noise, Wmu, bmu.reshape(1, z_dim),
      Wlv, blv.reshape(1, z_dim), Wd1, bd1.reshape(1, dec_h),
      Wd2, bd2.reshape(1, expr))

    return (xs, mu, lv)


# speedup vs baseline: 2.2813x; 1.5354x over previous
import functools

import jax
import jax.numpy as jnp
from jax import lax
from jax.experimental import pallas as pl
from jax.experimental.pallas import tpu as pltpu
from jax.experimental.pallas import tpu_sc as plsc

NC = 2
NS = 16
NW = NC * NS
CHUNK = 128
BLK = 1264

_P = jax.lax.Precision.HIGHEST


def _sc_mesh():
    return plsc.VectorSubcoreMesh(
        core_axis_name="c", subcore_axis_name="s", num_cores=NC,
        num_subcores=NS)


def _deg_body(npt, ch, sp_ref, rp_ref, ones_hbm, zero_hbm, out_ref,
              idx_s_v, idx_r_v, ones_v, degs_sp, degr_sp):
    c = lax.axis_index("c")
    s = lax.axis_index("s")
    wid = c * NS + s
    rows = pl.ds(s * npt, npt)
    pltpu.sync_copy(zero_hbm.at[rows], degs_sp.at[rows])
    pltpu.sync_copy(zero_hbm.at[rows], degr_sp.at[rows])
    pltpu.sync_copy(ones_hbm, ones_v)
    pltpu.sync_copy(sp_ref.at[wid], idx_s_v)
    pltpu.sync_copy(rp_ref.at[wid], idx_r_v)
    plsc.subcore_barrier()

    def body(j, carry):
        pltpu.sync_copy(ones_v, degs_sp.at[idx_s_v.at[j]], add=True)
        pltpu.sync_copy(ones_v, degr_sp.at[idx_r_v.at[j]], add=True)
        return carry

    lax.fori_loop(0, ch, body, 0)
    plsc.subcore_barrier()
    pltpu.sync_copy(degs_sp.at[rows], out_ref.at[c, 0, rows])
    pltpu.sync_copy(degr_sp.at[rows], out_ref.at[c, 1, rows])


def _prop_body(npt, ch, nb, table_hbm, sp_ref, rp_ref, zero_hbm, out_ref,
               idx_s_v, idx_r_v, rows_v, agg_sp, gsem0, gsem1, ssem0, ssem1):
    c = lax.axis_index("c")
    s = lax.axis_index("s")
    wid = c * NS + s
    rows = pl.ds(s * npt, npt)
    gsem = (gsem0, gsem1)
    ssem = (ssem0, ssem1)
    pltpu.sync_copy(zero_hbm.at[rows], agg_sp.at[rows])
    pltpu.sync_copy(sp_ref.at[wid], idx_s_v)
    pltpu.sync_copy(rp_ref.at[wid], idx_r_v)
    plsc.subcore_barrier()

    def issue_gathers(p, base):
        for b in range(nb):
            pltpu.async_copy(table_hbm.at[idx_s_v.at[base + b]],
                             rows_v.at[p, b], gsem[p])

    def drain_gathers(p):
        for b in range(nb):
            pltpu.make_async_copy(table_hbm.at[idx_s_v.at[0]],
                                  rows_v.at[p, b], gsem[p]).wait()

    def issue_scatters(p, base):
        for b in range(nb):
            pltpu.async_copy(rows_v.at[p, b], agg_sp.at[idx_r_v.at[base + b]],
                             ssem[p], add=True)

    def drain_scatters(p):
        for b in range(nb):
            pltpu.make_async_copy(rows_v.at[p, b], agg_sp.at[idx_r_v.at[0]],
                                  ssem[p]).wait()

    nk = ch // (2 * nb)
    issue_gathers(0, 0)

    def body(k, carry):
        base0 = k * 2 * nb
        base1 = base0 + nb
        drain_gathers(0)

        @pl.when(k > 0)
        def _():
            drain_scatters(1)

        issue_gathers(1, base1)
        issue_scatters(0, base0)
        drain_gathers(1)
        drain_scatters(0)

        @pl.when(k + 1 < nk)
        def _():
            issue_gathers(0, base0 + 2 * nb)

        issue_scatters(1, base1)
        return carry

    lax.fori_loop(0, nk, body, 0)
    drain_scatters(1)
    plsc.subcore_barrier()
    pltpu.sync_copy(agg_sp.at[rows], out_ref.at[c, rows])


def _scales(dp):
    ds_ = dp[0, 0, :, 0:1] + dp[1, 0, :, 0:1]
    dr_ = dp[0, 1, :, 0:1] + dp[1, 1, :, 0:1]
    ss = lax.rsqrt(jnp.maximum(ds_, 1.0))
    sr = lax.rsqrt(jnp.maximum(dr_, 1.0))
    return ss, sr


def _softmax(a):
    e = jnp.exp(a - jnp.max(a, axis=-1, keepdims=True))
    return e / jnp.sum(e, axis=-1, keepdims=True)


def _pad_cols(t, w):
    return jnp.concatenate(
        [t, jnp.zeros((t.shape[0], w - t.shape[1]), jnp.float32)], axis=1)


def _prep1_body(hp, x_ref, deg_ref, w_ref, b_ref, o_ref):
    ss, _ = _scales(deg_ref[...])
    a = jnp.maximum(
        jnp.dot(x_ref[...], w_ref[...], preferred_element_type=jnp.float32,
                precision=_P) + b_ref[...], 0.0)
    o_ref[...] = _pad_cols(_softmax(a) * ss, hp)


def _prep2_body(h, hp, agg_ref, deg_ref, w_ref, b_ref, o_ref):
    ss, sr = _scales(deg_ref[...])
    h1 = (agg_ref[0] + agg_ref[1])[:, :h] * sr
    a = jnp.maximum(
        jnp.dot(h1, w_ref[...], preferred_element_type=jnp.float32,
                precision=_P) + b_ref[...], 0.0)
    o_ref[...] = _pad_cols(_softmax(a) * ss, hp)


def _final_body(h, agg_ref, deg_ref, x_ref, nz_ref, wmu_ref, bmu_ref,
                wlv_ref, blv_ref, wd1_ref, bd1_ref, wd2_ref, bd2_ref,
                xs_ref, mu_ref, lv_ref):
    _, sr = _scales(deg_ref[...])
    h2 = (agg_ref[0] + agg_ref[1])[:, :h] * sr
    xb = x_ref[...]
    wmu = wmu_ref[...]
    wlv = wlv_ref[...]
    dot = functools.partial(jnp.dot, preferred_element_type=jnp.float32,
                            precision=_P)
    mu = dot(h2, wmu[:h]) + dot(xb, wmu[h:]) + bmu_ref[...]
    lv = dot(h2, wlv[:h]) + dot(xb, wlv[h:]) + blv_ref[...]
    sigma = 0.0001 + jnp.exp(0.5 * lv)
    z = mu + sigma * nz_ref[...]
    d = jnp.maximum(dot(z, wd1_ref[...]) + bd1_ref[...], 0.0)
    xs_ref[...] = dot(d, wd2_ref[...]) + bd2_ref[...]
    mu_ref[...] = mu
    lv_ref[...] = lv


def kernel(x, edge_index, noise, W1, b1, W2, b2, Wmu, bmu, Wlv, blv,
           Wd1, bd1, Wd2, bd2):
    f32 = jnp.float32
    n, d_feat = x.shape
    e = edge_index.shape[1]
    h = W1.shape[1]
    z_dim = Wmu.shape[1]
    dec_h = Wd1.shape[1]
    expr = Wd2.shape[1]

    hp = 32
    dw = 16
    nb = 4
    ch = 2 * nb * -(-e // (NW * CHUNK * 2 * nb))
    e_pad = NW * ch * CHUNK
    npad = BLK * (-(-(n + 1) // BLK))
    npt = npad // NS

    pad_ids = n + jnp.arange(e_pad - e, dtype=jnp.int32) % (npad - n)
    senders = jnp.concatenate(
        [edge_index[0], pad_ids]).reshape(NW, ch, CHUNK)
    receivers = jnp.concatenate(
        [edge_index[1], pad_ids]).reshape(NW, ch, CHUNK)

    zero_nd = jnp.zeros((npad, dw), f32)
    zero_nh = jnp.zeros((npad, hp), f32)
    ones_cd = jnp.ones((CHUNK, dw), f32)

    deg_part = pl.kernel(
        functools.partial(_deg_body, npt, ch),
        out_type=jax.ShapeDtypeStruct((NC, 2, npad, dw), f32),
        mesh=_sc_mesh(),
        scratch_types=[
            pltpu.VMEM((ch, CHUNK), jnp.int32),
            pltpu.VMEM((ch, CHUNK), jnp.int32),
            pltpu.VMEM((CHUNK, dw), f32),
            pltpu.VMEM_SHARED((npad, dw), f32),
            pltpu.VMEM_SHARED((npad, dw), f32),
        ],
        compiler_params=pltpu.CompilerParams(use_tc_tiling_on_sc=False),
    )(senders, receivers, ones_cd, zero_nd)

    grid = (npad // BLK,)
    row_spec = lambda w: pl.BlockSpec((BLK, w), lambda i: (i, 0))
    full_spec = lambda a, b: pl.BlockSpec((a, b), lambda i: (0, 0))
    agg_spec = pl.BlockSpec((NC, BLK, hp), lambda i: (0, i, 0))
    deg_spec = pl.BlockSpec((NC, 2, BLK, dw), lambda i: (0, 0, i, 0))

    table1 = pl.pallas_call(
        functools.partial(_prep1_body, hp),
        grid=grid,
        in_specs=[row_spec(d_feat), deg_spec, full_spec(d_feat, h),
                  full_spec(1, h)],
        out_specs=row_spec(hp),
        out_shape=jax.ShapeDtypeStruct((npad, hp), f32),
    )(x, deg_part, W1, b1.reshape(1, h))

    def propagate(table):
        return pl.kernel(
            functools.partial(_prop_body, npt, ch, nb),
            out_type=jax.ShapeDtypeStruct((NC, npad, hp), f32),
            mesh=_sc_mesh(),
            scratch_types=[
                pltpu.VMEM((ch, CHUNK), jnp.int32),
                pltpu.VMEM((ch, CHUNK), jnp.int32),
                pltpu.VMEM((2, nb, CHUNK, hp), f32),
                pltpu.VMEM_SHARED((npad, hp), f32),
                pltpu.SemaphoreType.DMA,
                pltpu.SemaphoreType.DMA,
                pltpu.SemaphoreType.DMA,
                pltpu.SemaphoreType.DMA,
            ],
            compiler_params=pltpu.CompilerParams(use_tc_tiling_on_sc=False),
        )(table, senders, receivers, zero_nh)

    agg1 = propagate(table1)

    table2 = pl.pallas_call(
        functools.partial(_prep2_body, h, hp),
        grid=grid,
        in_specs=[agg_spec, deg_spec, full_spec(h, h), full_spec(1, h)],
        out_specs=row_spec(hp),
        out_shape=jax.ShapeDtypeStruct((npad, hp), f32),
    )(agg1, deg_part, W2, b2.reshape(1, h))

    agg2 = propagate(table2)

    xs, mu, lv = pl.pallas_call(
        functools.partial(_final_body, h),
        grid=grid,
        in_specs=[agg_spec, deg_spec, row_spec(d_feat), row_spec(z_dim),
                  full_spec(h + d_feat, z_dim), full_spec(1, z_dim),
                  full_spec(h + d_feat, z_dim), full_spec(1, z_dim),
                  full_spec(z_dim, dec_h), full_spec(1, dec_h),
                  full_spec(dec_h, expr), full_spec(1, expr)],
        out_specs=[row_spec(expr), row_spec(z_dim), row_spec(z_dim)],
        out_shape=[
            jax.ShapeDtypeStruct((n, expr), f32),
            jax.ShapeDtypeStruct((n, z_dim), f32),
            jax.ShapeDtypeStruct((n, z_dim), f32),
        ],
    )(agg2, deg_part, x, noise, Wmu, bmu.reshape(1, z_dim),
      Wlv, blv.reshape(1, z_dim), Wd1, bd1.reshape(1, dec_h),
      Wd2, bd2.reshape(1, expr))

    return (xs, mu, lv)


# speedup vs baseline: 2.4039x; 1.0537x over previous
import functools

import jax
import jax.numpy as jnp
from jax import lax
from jax.experimental import pallas as pl
from jax.experimental.pallas import tpu as pltpu
from jax.experimental.pallas import tpu_sc as plsc

NC = 2
NS = 16
NW = NC * NS
CHUNK = 128
BLK = 1264

_P = jax.lax.Precision.HIGHEST


def _sc_mesh():
    return plsc.VectorSubcoreMesh(
        core_axis_name="c", subcore_axis_name="s", num_cores=NC,
        num_subcores=NS)


def _deg_body(npt, ch, sp_ref, rp_ref, ones_hbm, zero_hbm, out_ref,
              idx_s_v, idx_r_v, ones_v, degs_sp, degr_sp, sem):
    c = lax.axis_index("c")
    s = lax.axis_index("s")
    wid = c * NS + s
    rows = pl.ds(s * npt, npt)
    pltpu.sync_copy(zero_hbm.at[rows], degs_sp.at[rows])
    pltpu.sync_copy(zero_hbm.at[rows], degr_sp.at[rows])
    pltpu.sync_copy(ones_hbm, ones_v)
    pltpu.sync_copy(sp_ref.at[wid], idx_s_v)
    pltpu.sync_copy(rp_ref.at[wid], idx_r_v)
    plsc.subcore_barrier()

    def body(j, carry):
        pltpu.async_copy(ones_v, degs_sp.at[idx_s_v.at[j]], sem, add=True)
        pltpu.async_copy(ones_v, degr_sp.at[idx_r_v.at[j]], sem, add=True)
        return carry

    lax.fori_loop(0, ch, body, 0)

    def drain(j, carry):
        pltpu.make_async_copy(ones_v, degs_sp.at[idx_s_v.at[0]], sem).wait()
        pltpu.make_async_copy(ones_v, degr_sp.at[idx_r_v.at[0]], sem).wait()
        return carry

    lax.fori_loop(0, ch, drain, 0)
    plsc.subcore_barrier()
    pltpu.sync_copy(degs_sp.at[rows], out_ref.at[c, 0, rows])
    pltpu.sync_copy(degr_sp.at[rows], out_ref.at[c, 1, rows])


def _prop_body(npt, ch, nb, table_hbm, sp_ref, rp_ref, zero_hbm, out_ref,
               idx_s_v, idx_r_v, rows_v, agg_sp, gsem0, gsem1, ssem0, ssem1):
    c = lax.axis_index("c")
    s = lax.axis_index("s")
    wid = c * NS + s
    rows = pl.ds(s * npt, npt)
    gsem = (gsem0, gsem1)
    ssem = (ssem0, ssem1)
    pltpu.sync_copy(zero_hbm.at[rows], agg_sp.at[rows])
    pltpu.sync_copy(sp_ref.at[wid], idx_s_v)
    pltpu.sync_copy(rp_ref.at[wid], idx_r_v)
    plsc.subcore_barrier()

    def issue_gathers(p, base):
        for b in range(nb):
            pltpu.async_copy(table_hbm.at[idx_s_v.at[base + b]],
                             rows_v.at[p, b], gsem[p])

    def drain_gathers(p):
        for b in range(nb):
            pltpu.make_async_copy(table_hbm.at[idx_s_v.at[0]],
                                  rows_v.at[p, b], gsem[p]).wait()

    def issue_scatters(p, base):
        for b in range(nb):
            pltpu.async_copy(rows_v.at[p, b], agg_sp.at[idx_r_v.at[base + b]],
                             ssem[p], add=True)

    def drain_scatters(p):
        for b in range(nb):
            pltpu.make_async_copy(rows_v.at[p, b], agg_sp.at[idx_r_v.at[0]],
                                  ssem[p]).wait()

    nk = ch // (2 * nb)
    issue_gathers(0, 0)

    def body(k, carry):
        base0 = k * 2 * nb
        base1 = base0 + nb
        drain_gathers(0)

        @pl.when(k > 0)
        def _():
            drain_scatters(1)

        issue_gathers(1, base1)
        issue_scatters(0, base0)
        drain_gathers(1)
        drain_scatters(0)

        @pl.when(k + 1 < nk)
        def _():
            issue_gathers(0, base0 + 2 * nb)

        issue_scatters(1, base1)
        return carry

    lax.fori_loop(0, nk, body, 0)
    drain_scatters(1)
    plsc.subcore_barrier()
    pltpu.sync_copy(agg_sp.at[rows], out_ref.at[c, rows])


def _scales(dp):
    ds_ = dp[0, 0, :, 0:1] + dp[1, 0, :, 0:1]
    dr_ = dp[0, 1, :, 0:1] + dp[1, 1, :, 0:1]
    ss = lax.rsqrt(jnp.maximum(ds_, 1.0))
    sr = lax.rsqrt(jnp.maximum(dr_, 1.0))
    return ss, sr


def _softmax(a):
    e = jnp.exp(a - jnp.max(a, axis=-1, keepdims=True))
    return e / jnp.sum(e, axis=-1, keepdims=True)


def _pad_cols(t, w):
    return jnp.concatenate(
        [t, jnp.zeros((t.shape[0], w - t.shape[1]), jnp.float32)], axis=1)


def _prep1_body(hp, x_ref, deg_ref, w_ref, b_ref, o_ref):
    ss, _ = _scales(deg_ref[...])
    a = jnp.maximum(
        jnp.dot(x_ref[...], w_ref[...], preferred_element_type=jnp.float32,
                precision=_P) + b_ref[...], 0.0)
    o_ref[...] = _pad_cols(_softmax(a) * ss, hp)


def _prep2_body(h, hp, agg_ref, deg_ref, w_ref, b_ref, o_ref):
    ss, sr = _scales(deg_ref[...])
    h1 = (agg_ref[0] + agg_ref[1])[:, :h] * sr
    a = jnp.maximum(
        jnp.dot(h1, w_ref[...], preferred_element_type=jnp.float32,
                precision=_P) + b_ref[...], 0.0)
    o_ref[...] = _pad_cols(_softmax(a) * ss, hp)


def _final_body(h, agg_ref, deg_ref, x_ref, nz_ref, wmu_ref, bmu_ref,
                wlv_ref, blv_ref, wd1_ref, bd1_ref, wd2_ref, bd2_ref,
                xs_ref, mu_ref, lv_ref):
    _, sr = _scales(deg_ref[...])
    h2 = (agg_ref[0] + agg_ref[1])[:, :h] * sr
    xb = x_ref[...]
    wmu = wmu_ref[...]
    wlv = wlv_ref[...]
    dot = functools.partial(jnp.dot, preferred_element_type=jnp.float32,
                            precision=_P)
    mu = dot(h2, wmu[:h]) + dot(xb, wmu[h:]) + bmu_ref[...]
    lv = dot(h2, wlv[:h]) + dot(xb, wlv[h:]) + blv_ref[...]
    sigma = 0.0001 + jnp.exp(0.5 * lv)
    z = mu + sigma * nz_ref[...]
    d = jnp.maximum(dot(z, wd1_ref[...]) + bd1_ref[...], 0.0)
    xs_ref[...] = dot(d, wd2_ref[...]) + bd2_ref[...]
    mu_ref[...] = mu
    lv_ref[...] = lv


def kernel(x, edge_index, noise, W1, b1, W2, b2, Wmu, bmu, Wlv, blv,
           Wd1, bd1, Wd2, bd2):
    f32 = jnp.float32
    n, d_feat = x.shape
    e = edge_index.shape[1]
    h = W1.shape[1]
    z_dim = Wmu.shape[1]
    dec_h = Wd1.shape[1]
    expr = Wd2.shape[1]

    hp = 32
    dw = 16
    nb = 8
    ch = 2 * nb * -(-e // (NW * CHUNK * 2 * nb))
    e_pad = NW * ch * CHUNK
    npad = BLK * (-(-(n + 1) // BLK))
    npt = npad // NS

    pad_ids = n + jnp.arange(e_pad - e, dtype=jnp.int32) % (npad - n)
    senders = jnp.concatenate(
        [edge_index[0], pad_ids]).reshape(NW, ch, CHUNK)
    receivers = jnp.concatenate(
        [edge_index[1], pad_ids]).reshape(NW, ch, CHUNK)

    zero_nd = jnp.zeros((npad, dw), f32)
    zero_nh = jnp.zeros((npad, hp), f32)
    ones_cd = jnp.ones((CHUNK, dw), f32)

    deg_part = pl.kernel(
        functools.partial(_deg_body, npt, ch),
        out_type=jax.ShapeDtypeStruct((NC, 2, npad, dw), f32),
        mesh=_sc_mesh(),
        scratch_types=[
            pltpu.VMEM((ch, CHUNK), jnp.int32),
            pltpu.VMEM((ch, CHUNK), jnp.int32),
            pltpu.VMEM((CHUNK, dw), f32),
            pltpu.VMEM_SHARED((npad, dw), f32),
            pltpu.VMEM_SHARED((npad, dw), f32),
            pltpu.SemaphoreType.DMA,
        ],
        compiler_params=pltpu.CompilerParams(use_tc_tiling_on_sc=False),
    )(senders, receivers, ones_cd, zero_nd)

    grid = (npad // BLK,)
    row_spec = lambda w: pl.BlockSpec((BLK, w), lambda i: (i, 0))
    full_spec = lambda a, b: pl.BlockSpec((a, b), lambda i: (0, 0))
    agg_spec = pl.BlockSpec((NC, BLK, hp), lambda i: (0, i, 0))
    deg_spec = pl.BlockSpec((NC, 2, BLK, dw), lambda i: (0, 0, i, 0))

    table1 = pl.pallas_call(
        functools.partial(_prep1_body, hp),
        grid=grid,
        in_specs=[row_spec(d_feat), deg_spec, full_spec(d_feat, h),
                  full_spec(1, h)],
        out_specs=row_spec(hp),
        out_shape=jax.ShapeDtypeStruct((npad, hp), f32),
    )(x, deg_part, W1, b1.reshape(1, h))

    def propagate(table):
        return pl.kernel(
            functools.partial(_prop_body, npt, ch, nb),
            out_type=jax.ShapeDtypeStruct((NC, npad, hp), f32),
            mesh=_sc_mesh(),
            scratch_types=[
                pltpu.VMEM((ch, CHUNK), jnp.int32),
                pltpu.VMEM((ch, CHUNK), jnp.int32),
                pltpu.VMEM((2, nb, CHUNK, hp), f32),
                pltpu.VMEM_SHARED((npad, hp), f32),
                pltpu.SemaphoreType.DMA,
                pltpu.SemaphoreType.DMA,
                pltpu.SemaphoreType.DMA,
                pltpu.SemaphoreType.DMA,
            ],
            compiler_params=pltpu.CompilerParams(use_tc_tiling_on_sc=False),
        )(table, senders, receivers, zero_nh)

    agg1 = propagate(table1)

    table2 = pl.pallas_call(
        functools.partial(_prep2_body, h, hp),
        grid=grid,
        in_specs=[agg_spec, deg_spec, full_spec(h, h), full_spec(1, h)],
        out_specs=row_spec(hp),
        out_shape=jax.ShapeDtypeStruct((npad, hp), f32),
    )(agg1, deg_part, W2, b2.reshape(1, h))

    agg2 = propagate(table2)

    xs, mu, lv = pl.pallas_call(
        functools.partial(_final_body, h),
        grid=grid,
        in_specs=[agg_spec, deg_spec, row_spec(d_feat), row_spec(z_dim),
                  full_spec(h + d_feat, z_dim), full_spec(1, z_dim),
                  full_spec(h + d_feat, z_dim), full_spec(1, z_dim),
                  full_spec(z_dim, dec_h), full_spec(1, dec_h),
                  full_spec(dec_h, expr), full_spec(1, expr)],
        out_specs=[row_spec(expr), row_spec(z_dim), row_spec(z_dim)],
        out_shape=[
            jax.ShapeDtypeStruct((n, expr), f32),
            jax.ShapeDtypeStruct((n, z_dim), f32),
            jax.ShapeDtypeStruct((n, z_dim), f32),
        ],
    )(agg2, deg_part, x, noise, Wmu, bmu.reshape(1, z_dim),
      Wlv, blv.reshape(1, z_dim), Wd1, bd1.reshape(1, dec_h),
      Wd2, bd2.reshape(1, expr))

    return (xs, mu, lv)
